# baseline (speedup 1.0000x reference)
import functools

import jax
import jax.numpy as jnp
from jax import lax
from jax.experimental import pallas as pl
from jax.experimental.pallas import tpu as pltpu
from jax.experimental.pallas import tpu_sc as plsc

VOCAB = 50257
N_EMBD = 768
BATCH = 4
SEQ = 2048
TOKENS = BATCH * SEQ
NUM_CORES = 2
NUM_SUBCORES = 16
NW = NUM_CORES * NUM_SUBCORES
SEQ_BLK = SEQ // NW
SUB = 32
N_SUB = SEQ_BLK // SUB
N_CH = BATCH * N_SUB
NBUF = 3
LANES = 16
SLICES = N_EMBD // LANES


def _sc_body(ids_hbm, wte_hbm, wpe_hbm, out_hbm,
             idx_v, wpe_v, wte_v0, wte_v1, wte_v2,
             gs0, gs1, gs2, os0, os1, os2, ws0, ws1, isem):
    wid = lax.axis_index("s") * NUM_CORES + lax.axis_index("c")
    sb = wid * SEQ_BLK
    wte_bufs = (wte_v0, wte_v1, wte_v2)
    gsems = (gs0, gs1, gs2)
    osems = (os0, os1, os2)

    half = SEQ_BLK // 2
    wpe_cp0 = pltpu.async_copy(wpe_hbm.at[pl.ds(sb, half)],
                               wpe_v.at[pl.ds(0, half)], ws0)
    id_copies = [
        pltpu.async_copy(ids_hbm.at[b, pl.ds(sb, SEQ_BLK)],
                         idx_v.at[pl.ds(b * SEQ_BLK, SEQ_BLK)], isem)
        for b in range(BATCH)
    ]
    wpe_cp1 = pltpu.async_copy(wpe_hbm.at[pl.ds(sb + half, half)],
                               wpe_v.at[pl.ds(half, half)], ws1)
    for cp in id_copies:
        cp.wait()

    def start_gather(c):
        p = c % NBUF
        return pltpu.async_copy(
            wte_hbm.at[idx_v.at[pl.ds(c * SUB, SUB)]], wte_bufs[p], gsems[p])

    gathers = [None] * N_CH
    stores = [None] * N_CH
    gathers[0] = start_gather(0)
    wpe_cp0.wait()
    for c in range(N_CH):
        p = c % NBUF
        b, h = c // N_SUB, c % N_SUB
        nc = c + 1
        if nc < N_CH:
            if nc >= NBUF:
                stores[nc - NBUF].wait()
            gathers[nc] = start_gather(nc)
        if c == 1:
            wpe_cp1.wait()
        gathers[c].wait()
        wte_buf = wte_bufs[p]

        @plsc.parallel_loop(0, SUB, 1, unroll=4)
        def row_add(r):
            for j in range(SLICES):
                sl = pl.ds(j * LANES, LANES)
                plsc.addupdate(wte_buf.at[r, sl], wpe_v[h * SUB + r, sl])

        stores[c] = pltpu.async_copy(
            wte_buf, out_hbm.at[b, pl.ds(sb + h * SUB, SUB)], osems[p])
    for c in range(N_CH - NBUF, N_CH):
        stores[c].wait()


def _make_sc_kernel():
    return functools.partial(
        pl.kernel,
        mesh=plsc.VectorSubcoreMesh(core_axis_name="c", subcore_axis_name="s"),
        out_type=jax.ShapeDtypeStruct((BATCH, SEQ, N_EMBD), jnp.float32),
        scratch_types=(
            [pltpu.VMEM((BATCH * SEQ_BLK,), jnp.int32),
             pltpu.VMEM((SEQ_BLK, N_EMBD), jnp.float32)]
            + [pltpu.VMEM((SUB, N_EMBD), jnp.float32)] * NBUF
            + [pltpu.SemaphoreType.DMA] * (2 * NBUF + 3)
        ),
    )(_sc_body)


_sc_kernel = None


def kernel(input_ids, wte, wpe):
    global _sc_kernel
    if _sc_kernel is None:
        _sc_kernel = _make_sc_kernel()
    return _sc_kernel(input_ids, wte, wpe)


# speedup vs baseline: 1.1306x; 1.1306x over previous
import functools

import jax
import jax.numpy as jnp
from jax import lax
from jax.experimental import pallas as pl
from jax.experimental.pallas import tpu as pltpu
from jax.experimental.pallas import tpu_sc as plsc

VOCAB = 50257
N_EMBD = 768
BATCH = 4
SEQ = 2048
TOKENS = BATCH * SEQ
NUM_CORES = 2
NUM_SUBCORES = 16
NW = NUM_CORES * NUM_SUBCORES
SEQ_BLK = SEQ // NW
POS_CH = 8
SUB = BATCH * POS_CH
N_CH = SEQ_BLK // POS_CH
NBUF = 3
LANES = 16
SLICES = N_EMBD // LANES


def _sc_body(ids_hbm, wte_hbm, wpe_hbm, out_hbm,
             idx_v, wpe_v, wte_v0, wte_v1, wte_v2,
             gs0, gs1, gs2, os0, os1, os2, ws0, ws1, isem):
    wid = lax.axis_index("s") * NUM_CORES + lax.axis_index("c")
    sb = wid * SEQ_BLK
    wte_bufs = (wte_v0, wte_v1, wte_v2)
    gsems = (gs0, gs1, gs2)
    osems = (os0, os1, os2)

    half = SEQ_BLK // 2
    wpe_cp0 = pltpu.async_copy(wpe_hbm.at[pl.ds(sb, half)],
                               wpe_v.at[pl.ds(0, half)], ws0)
    id_copies = [
        pltpu.async_copy(
            ids_hbm.at[b, pl.ds(sb + k * POS_CH, POS_CH)],
            idx_v.at[pl.ds(k * SUB + b * POS_CH, POS_CH)], isem)
        for k in range(N_CH) for b in range(BATCH)
    ]
    wpe_cp1 = pltpu.async_copy(wpe_hbm.at[pl.ds(sb + half, half)],
                               wpe_v.at[pl.ds(half, half)], ws1)
    for cp in id_copies:
        cp.wait()

    def start_gather(c):
        p = c % NBUF
        return pltpu.async_copy(
            wte_hbm.at[idx_v.at[pl.ds(c * SUB, SUB)]], wte_bufs[p], gsems[p])

    gathers = [None] * N_CH
    stores = [[None] * BATCH for _ in range(N_CH)]
    gathers[0] = start_gather(0)
    wpe_cp0.wait()
    for c in range(N_CH):
        p = c % NBUF
        nc = c + 1
        if nc < N_CH:
            if nc >= NBUF:
                for s in stores[nc - NBUF]:
                    s.wait()
            gathers[nc] = start_gather(nc)
        if c == N_CH // 2:
            wpe_cp1.wait()
        gathers[c].wait()
        wte_buf = wte_bufs[p]

        @plsc.parallel_loop(0, POS_CH, 1, unroll=1)
        def pos_add(i):
            for j in range(SLICES):
                sl = pl.ds(j * LANES, LANES)
                w = wpe_v[c * POS_CH + i, sl]
                for b in range(BATCH):
                    plsc.addupdate(wte_buf.at[b * POS_CH + i, sl], w)

        for b in range(BATCH):
            stores[c][b] = pltpu.async_copy(
                wte_buf.at[pl.ds(b * POS_CH, POS_CH)],
                out_hbm.at[b, pl.ds(sb + c * POS_CH, POS_CH)], osems[p])
    for c in range(N_CH - NBUF, N_CH):
        for s in stores[c]:
            s.wait()


def _make_sc_kernel():
    return functools.partial(
        pl.kernel,
        mesh=plsc.VectorSubcoreMesh(core_axis_name="c", subcore_axis_name="s"),
        out_type=jax.ShapeDtypeStruct((BATCH, SEQ, N_EMBD), jnp.float32),
        scratch_types=(
            [pltpu.VMEM((SEQ_BLK * BATCH,), jnp.int32),
             pltpu.VMEM((SEQ_BLK, N_EMBD), jnp.float32)]
            + [pltpu.VMEM((SUB, N_EMBD), jnp.float32)] * NBUF
            + [pltpu.SemaphoreType.DMA] * (2 * NBUF + 3)
        ),
    )(_sc_body)


_sc_kernel = None


def kernel(input_ids, wte, wpe):
    global _sc_kernel
    if _sc_kernel is None:
        _sc_kernel = _make_sc_kernel()
    return _sc_kernel(input_ids, wte, wpe)
